# Initial kernel scaffold; baseline (speedup 1.0000x reference)
import jax
import jax.numpy as jnp
from jax.experimental import pallas as pl


def kernel(tokens, table):
    raise NotImplementedError("write your pallas kernel here")


# speedup vs baseline: 3.2623x; 3.2623x over previous
import functools

import jax
import jax.numpy as jnp
from jax import lax
from jax.experimental import pallas as pl
from jax.experimental.pallas import tpu as pltpu
from jax.experimental.pallas import tpu_sc as plsc

_EMB = 64
_SCALE = 8.0

_NC, _NS = 2, 16
_NW = _NC * _NS
_CHUNK = 128


def _scale_body(t_ref, o_ref):
    o_ref[...] = t_ref[...] * _SCALE


def _scale_table(table):
    rows = table.shape[0]
    block = 2000
    assert rows % block == 0
    return pl.pallas_call(
        _scale_body,
        out_shape=jax.ShapeDtypeStruct(table.shape, table.dtype),
        grid=(rows // block,),
        in_specs=[pl.BlockSpec((block, _EMB), lambda i: (i, 0))],
        out_specs=pl.BlockSpec((block, _EMB), lambda i: (i, 0)),
    )(table)


def _gather_body(n_chunks, table_hbm, tok_hbm, out_hbm, idx_v, rows_v, gsem):
    w = lax.axis_index("s") * _NC + lax.axis_index("c")
    pltpu.sync_copy(tok_hbm.at[pl.ds(w * n_chunks, n_chunks)], idx_v)
    row_base = w * n_chunks * _CHUNK

    def chunk(j, carry):
        pltpu.async_copy(table_hbm.at[idx_v.at[j]], rows_v, gsem).wait()
        pltpu.sync_copy(rows_v, out_hbm.at[pl.ds(row_base + j * _CHUNK, _CHUNK)])
        return carry

    lax.fori_loop(0, n_chunks, chunk, 0)


def kernel(tokens, table):
    orig_shape = tokens.shape
    flat = tokens.reshape(-1).astype(jnp.int32)
    total = flat.shape[0]
    assert total % (_NW * _CHUNK) == 0
    n_chunks = total // (_NW * _CHUNK)
    tok2d = flat.reshape(total // _CHUNK, _CHUNK)

    scaled = _scale_table(table)

    mesh = plsc.VectorSubcoreMesh(core_axis_name="c", subcore_axis_name="s")
    out = pl.kernel(
        functools.partial(_gather_body, n_chunks),
        out_type=jax.ShapeDtypeStruct((total, _EMB), jnp.float32),
        mesh=mesh,
        compiler_params=pltpu.CompilerParams(use_tc_tiling_on_sc=False),
        scratch_types=[
            pltpu.VMEM((n_chunks, _CHUNK), jnp.int32),
            pltpu.VMEM((_CHUNK, _EMB), jnp.float32),
            pltpu.SemaphoreType.DMA,
        ],
    )(scaled, tok2d)
    return out.reshape(*orig_shape, _EMB)


# speedup vs baseline: 3.8702x; 1.1863x over previous
import functools

import jax
import jax.numpy as jnp
from jax import lax
from jax.experimental import pallas as pl
from jax.experimental.pallas import tpu as pltpu
from jax.experimental.pallas import tpu_sc as plsc

_EMB = 64
_SCALE = 8.0

_NC, _NS = 2, 16
_NW = _NC * _NS
_CHUNK = 128


def _scale_body(t_ref, o_ref):
    o_ref[...] = t_ref[...] * _SCALE


def _scale_table(table):
    rows = table.shape[0]
    block = 2000
    assert rows % block == 0
    return pl.pallas_call(
        _scale_body,
        out_shape=jax.ShapeDtypeStruct(table.shape, table.dtype),
        grid=(rows // block,),
        in_specs=[pl.BlockSpec((block, _EMB), lambda i: (i, 0))],
        out_specs=pl.BlockSpec((block, _EMB), lambda i: (i, 0)),
    )(table)


_NBUF = 8


def _gather_body(n_chunks, table_hbm, tok_hbm, out_hbm, idx_v, rows_v, gsem, osem):
    w = lax.axis_index("s") * _NC + lax.axis_index("c")
    pltpu.sync_copy(tok_hbm.at[pl.ds(w * n_chunks, n_chunks)], idx_v)
    row_base = w * n_chunks * _CHUNK
    n_groups = n_chunks // _NBUF

    def out_slice(j):
        return out_hbm.at[pl.ds(row_base + j * _CHUNK, _CHUNK)]

    for b in range(_NBUF):
        pltpu.async_copy(table_hbm.at[idx_v.at[b]], rows_v.at[b], gsem.at[b])

    def group(g, carry):
        for b in range(_NBUF):
            j = g * _NBUF + b
            pltpu.make_async_copy(
                table_hbm.at[idx_v.at[j]], rows_v.at[b], gsem.at[b]
            ).wait()
            pltpu.async_copy(rows_v.at[b], out_slice(j), osem.at[b])

            @pl.when(g + 1 < n_groups)
            def _():
                pltpu.make_async_copy(rows_v.at[b], out_slice(j), osem.at[b]).wait()
                pltpu.async_copy(
                    table_hbm.at[idx_v.at[j + _NBUF]], rows_v.at[b], gsem.at[b]
                )

        return carry

    lax.fori_loop(0, n_groups, group, 0)

    for b in range(_NBUF):
        j = n_chunks - _NBUF + b
        pltpu.make_async_copy(rows_v.at[b], out_slice(j), osem.at[b]).wait()


def kernel(tokens, table):
    orig_shape = tokens.shape
    flat = tokens.reshape(-1).astype(jnp.int32)
    total = flat.shape[0]
    assert total % (_NW * _CHUNK) == 0
    n_chunks = total // (_NW * _CHUNK)
    tok2d = flat.reshape(total // _CHUNK, _CHUNK)

    scaled = _scale_table(table)

    mesh = plsc.VectorSubcoreMesh(core_axis_name="c", subcore_axis_name="s")
    out = pl.kernel(
        functools.partial(_gather_body, n_chunks),
        out_type=jax.ShapeDtypeStruct((total, _EMB), jnp.float32),
        mesh=mesh,
        compiler_params=pltpu.CompilerParams(use_tc_tiling_on_sc=False),
        scratch_types=[
            pltpu.VMEM((n_chunks, _CHUNK), jnp.int32),
            pltpu.VMEM((_NBUF, _CHUNK, _EMB), jnp.float32),
            pltpu.SemaphoreType.DMA((_NBUF,)),
            pltpu.SemaphoreType.DMA((_NBUF,)),
        ],
    )(scaled, tok2d)
    return out.reshape(*orig_shape, _EMB)
